# baseline (speedup 1.0000x reference)
import functools

import jax
import jax.numpy as jnp
from jax import lax
from jax.experimental import pallas as pl
from jax.experimental.pallas import tpu as pltpu
from jax.experimental.pallas import tpu_sc as plsc

S, D, FF, E, K = 2048, 768, 3072, 8, 2
P = S * K
CBLK = 256
NB = 24
NSLOT = NB * CBLK
RB = 512
_SQRT2 = 1.4142135623730951
L = 16

_mesh = plsc.VectorSubcoreMesh(core_axis_name="c", subcore_axis_name="s")


def _router_body(x_ref, wr_ref, wts_ref, dest_ref, bm_ref):
    logits = jnp.dot(x_ref[...], wr_ref[...],
                     preferred_element_type=jnp.float32)
    m = jnp.max(logits, axis=-1, keepdims=True)
    p = jnp.exp(logits - m)
    p = p / jnp.sum(p, axis=-1, keepdims=True)
    p1 = jnp.max(p, axis=-1, keepdims=True)
    is1 = p == p1
    pm = jnp.where(is1, -1.0, p)
    p2 = jnp.max(pm, axis=-1, keepdims=True)
    is2 = pm == p2
    denom = p1 + p2
    wts_ref[...] = jnp.concatenate([p1 / denom, p2 / denom], axis=1)

    o1 = is1.astype(jnp.float32)
    o2 = is2.astype(jnp.float32)
    r = lax.broadcasted_iota(jnp.int32, (RB, RB), 0)
    c = lax.broadcasted_iota(jnp.int32, (RB, RB), 1)
    tril = (c < r).astype(jnp.float32)
    carry = jnp.zeros((1, E), jnp.float32)
    ranks = []
    for blk in range(P // RB):
        k_lvl, row0 = divmod(blk * RB, S)
        o_blk = (o1 if k_lvl == 0 else o2)[row0:row0 + RB, :]
        r_blk = jnp.dot(tril, o_blk, preferred_element_type=jnp.float32)
        r_blk = r_blk + carry
        carry = carry + jnp.sum(o_blk, axis=0, keepdims=True)
        ranks.append(jnp.sum(r_blk * o_blk, axis=1, keepdims=True))

    tot_i = carry.astype(jnp.int32)
    ptot_i = ((tot_i + (CBLK - 1)) // CBLK) * CBLK
    er = lax.broadcasted_iota(jnp.int32, (E, E), 0)
    ec = lax.broadcasted_iota(jnp.int32, (E, E), 1)
    etril = (er < ec).astype(jnp.float32)
    off = jnp.dot(ptot_i.astype(jnp.float32), etril,
                  preferred_element_type=jnp.float32)

    for k_lvl in range(K):
        o_full = o1 if k_lvl == 0 else o2
        off_g = jnp.sum(off * o_full, axis=1, keepdims=True)
        rk = jnp.concatenate(
            [ranks[k_lvl * (S // RB) + i] for i in range(S // RB)], axis=0)
        dest_ref[:, k_lvl:k_lvl + 1] = (off_g + rk).astype(jnp.int32)

    end = off + ptot_i.astype(jnp.float32)
    bstart = (lax.broadcasted_iota(jnp.int32, (64, 1), 0) * CBLK)
    emap = jnp.sum((bstart.astype(jnp.float32) >= end).astype(jnp.int32),
                   axis=1, keepdims=True)
    nlive = jnp.sum(ptot_i, axis=1, keepdims=True) // CBLK
    lanei = lax.broadcasted_iota(jnp.int32, (64, 1), 0)
    bm_ref[...] = jnp.where(lanei == 32, nlive,
                            jnp.minimum(emap, E - 1))


def _router(x, wr):
    return pl.pallas_call(
        _router_body,
        out_shape=[jax.ShapeDtypeStruct((S, K), jnp.float32),
                   jax.ShapeDtypeStruct((S, K), jnp.int32),
                   jax.ShapeDtypeStruct((64, 1), jnp.int32)],
    )(x, wr)


_SP = P // 32


def _scatter_body(dest_ref, wf_ref, srctok_ref, gate_ref, d_v, w_v, t_v, sem):
    cid = lax.axis_index("c")
    sid = lax.axis_index("s")
    w = sid * 2 + cid
    base = w * _SP
    pltpu.sync_copy(dest_ref.at[pl.ds(base, _SP)], d_v)
    pltpu.sync_copy(wf_ref.at[pl.ds(base, _SP)], w_v)
    lanes = lax.iota(jnp.int32, L)
    for j in range(_SP // L):
        tok = lax.shift_right_logical(base + j * L + lanes, 1)
        t_v[pl.ds(j * L, L)] = tok
    pltpu.async_copy(t_v, srctok_ref.at[d_v], sem).wait()
    pltpu.async_copy(w_v, gate_ref.at[d_v], sem).wait()


_scatter = functools.partial(
    pl.kernel,
    out_type=[jax.ShapeDtypeStruct((NSLOT,), jnp.int32),
              jax.ShapeDtypeStruct((NSLOT,), jnp.float32)],
    mesh=_mesh,
    scratch_types=[pltpu.VMEM((_SP,), jnp.int32),
                   pltpu.VMEM((_SP,), jnp.float32),
                   pltpu.VMEM((_SP,), jnp.int32),
                   pltpu.SemaphoreType.DMA],
)(_scatter_body)


_GSLOT = NSLOT // 32
_GC = 96
_GCH = _GSLOT // _GC


def _gather_body(srctok_ref, x_ref, xr_ref, idx_v, rows_v, sem):
    cid = lax.axis_index("c")
    sid = lax.axis_index("s")
    w = sid * 2 + cid
    for c in range(_GCH):
        start = w * _GSLOT + c * _GC
        pltpu.sync_copy(srctok_ref.at[pl.ds(start, _GC)], idx_v)
        for l in range(_GC // L):
            idx_v[pl.ds(l * L, L)] = idx_v[pl.ds(l * L, L)] & (S - 1)
        pltpu.async_copy(x_ref.at[idx_v], rows_v, sem).wait()
        pltpu.sync_copy(rows_v, xr_ref.at[pl.ds(start, _GC)])


_gather = functools.partial(
    pl.kernel,
    out_type=jax.ShapeDtypeStruct((NSLOT, D), jnp.float32),
    mesh=_mesh,
    scratch_types=[pltpu.VMEM((_GC,), jnp.int32),
                   pltpu.VMEM((_GC, D), jnp.float32),
                   pltpu.SemaphoreType.DMA],
)(_gather_body)


def _ffn_body(bm_ref, xr_ref, g_ref, w1_ref, b1_ref, w2_ref, b2_ref, y_ref):
    b = pl.program_id(0)

    @pl.when(b < bm_ref[32])
    def _go():
        h = jnp.dot(xr_ref[...], w1_ref[0],
                    preferred_element_type=jnp.float32) + b1_ref[0]
        h = 0.5 * h * (1.0 + lax.erf(h / _SQRT2))
        y = jnp.dot(h, w2_ref[0], preferred_element_type=jnp.float32)
        y_ref[...] = g_ref[...] * (y + b2_ref[0])


def _ffn(bm, xr, gates, w1, b1, w2, b2):
    grid_spec = pltpu.PrefetchScalarGridSpec(
        num_scalar_prefetch=1,
        grid=(NB,),
        in_specs=[
            pl.BlockSpec((CBLK, D), lambda b, bm: (b, 0)),
            pl.BlockSpec((CBLK, 1), lambda b, bm: (b, 0)),
            pl.BlockSpec((1, D, FF), lambda b, bm: (bm[b], 0, 0)),
            pl.BlockSpec((1, 1, FF), lambda b, bm: (bm[b], 0, 0)),
            pl.BlockSpec((1, FF, D), lambda b, bm: (bm[b], 0, 0)),
            pl.BlockSpec((1, 1, D), lambda b, bm: (bm[b], 0, 0)),
        ],
        out_specs=pl.BlockSpec((CBLK, D), lambda b, bm: (b, 0)),
    )
    return pl.pallas_call(
        _ffn_body,
        grid_spec=grid_spec,
        out_shape=jax.ShapeDtypeStruct((NSLOT, D), jnp.float32),
    )(bm, xr, gates, w1, b1.reshape(E, 1, FF), w2, b2.reshape(E, 1, D))


_CT = S // 32
_CC = 32


def _combine_body(pos_ref, y_ref, out_ref, idx_v, rows_v, o_v, sem):
    cid = lax.axis_index("c")
    sid = lax.axis_index("s")
    w = sid * 2 + cid
    for c in range(_CT // _CC):
        t0 = w * _CT + c * _CC
        pltpu.sync_copy(pos_ref.at[pl.ds(t0 * K, _CC * K)], idx_v)
        pltpu.async_copy(y_ref.at[idx_v], rows_v, sem).wait()

        def _row(j, _):
            for l in range(D // L):
                a = rows_v[2 * j, pl.ds(l * L, L)]
                b = rows_v[2 * j + 1, pl.ds(l * L, L)]
                o_v[j, pl.ds(l * L, L)] = a + b
            return 0

        lax.fori_loop(0, _CC, _row, 0)
        pltpu.sync_copy(o_v, out_ref.at[pl.ds(t0, _CC)])


_combine = functools.partial(
    pl.kernel,
    out_type=jax.ShapeDtypeStruct((S, D), jnp.float32),
    mesh=_mesh,
    scratch_types=[pltpu.VMEM((_CC * K,), jnp.int32),
                   pltpu.VMEM((_CC * K, D), jnp.float32),
                   pltpu.VMEM((_CC, D), jnp.float32),
                   pltpu.SemaphoreType.DMA],
)(_combine_body)


def kernel(hidden_states, W_router, W1, b1, W2, b2):
    x = hidden_states.reshape(S, D)
    wts, dest, bm = _router(x, W_router)
    destf = dest.reshape(P)
    srctok, gate = _scatter(destf, wts.reshape(P))
    xr = _gather(srctok, x)
    y = _ffn(bm.reshape(64), xr, gate.reshape(NSLOT, 1), W1, b1, W2, b2)
    out = _combine(destf, y)
    return out.reshape(1, S, D)


# speedup vs baseline: 1.6389x; 1.6389x over previous
import functools

import jax
import jax.numpy as jnp
from jax import lax
from jax.experimental import pallas as pl
from jax.experimental.pallas import tpu as pltpu
from jax.experimental.pallas import tpu_sc as plsc

S, D, FF, E, K = 2048, 768, 3072, 8, 2
P = S * K
CBLK = 256
NB = 24
NSLOT = NB * CBLK
RB = 512
_SQRT2 = 1.4142135623730951
L = 16

_mesh = plsc.VectorSubcoreMesh(core_axis_name="c", subcore_axis_name="s")


def _router_body(x_ref, wr_ref, wts_ref, dest_ref, bm_ref):
    logits = jnp.dot(x_ref[...], wr_ref[...],
                     preferred_element_type=jnp.float32)
    m = jnp.max(logits, axis=-1, keepdims=True)
    p = jnp.exp(logits - m)
    p = p / jnp.sum(p, axis=-1, keepdims=True)
    p1 = jnp.max(p, axis=-1, keepdims=True)
    is1 = p == p1
    pm = jnp.where(is1, -1.0, p)
    p2 = jnp.max(pm, axis=-1, keepdims=True)
    is2 = pm == p2
    denom = p1 + p2
    wts_ref[...] = jnp.concatenate([p1 / denom, p2 / denom], axis=1)

    o1 = is1.astype(jnp.float32)
    o2 = is2.astype(jnp.float32)
    r = lax.broadcasted_iota(jnp.int32, (RB, RB), 0)
    c = lax.broadcasted_iota(jnp.int32, (RB, RB), 1)
    tril = (c < r).astype(jnp.float32)
    carry = jnp.zeros((1, E), jnp.float32)
    ranks = []
    for blk in range(P // RB):
        k_lvl, row0 = divmod(blk * RB, S)
        o_blk = (o1 if k_lvl == 0 else o2)[row0:row0 + RB, :]
        r_blk = jnp.dot(tril, o_blk, preferred_element_type=jnp.float32)
        r_blk = r_blk + carry
        carry = carry + jnp.sum(o_blk, axis=0, keepdims=True)
        ranks.append(jnp.sum(r_blk * o_blk, axis=1, keepdims=True))

    tot_i = carry.astype(jnp.int32)
    ptot_i = ((tot_i + (CBLK - 1)) // CBLK) * CBLK
    er = lax.broadcasted_iota(jnp.int32, (E, E), 0)
    ec = lax.broadcasted_iota(jnp.int32, (E, E), 1)
    etril = (er < ec).astype(jnp.float32)
    off = jnp.dot(ptot_i.astype(jnp.float32), etril,
                  preferred_element_type=jnp.float32)

    for k_lvl in range(K):
        o_full = o1 if k_lvl == 0 else o2
        off_g = jnp.sum(off * o_full, axis=1, keepdims=True)
        rk = jnp.concatenate(
            [ranks[k_lvl * (S // RB) + i] for i in range(S // RB)], axis=0)
        dest_ref[:, k_lvl:k_lvl + 1] = (off_g + rk).astype(jnp.int32)

    end = off + ptot_i.astype(jnp.float32)
    bstart = (lax.broadcasted_iota(jnp.int32, (64, 1), 0) * CBLK)
    emap = jnp.sum((bstart.astype(jnp.float32) >= end).astype(jnp.int32),
                   axis=1, keepdims=True)
    nlive = jnp.sum(ptot_i, axis=1, keepdims=True) // CBLK
    lanei = lax.broadcasted_iota(jnp.int32, (64, 1), 0)
    bm_ref[...] = jnp.where(lanei == 32, nlive,
                            jnp.minimum(emap, E - 1))


def _router(x, wr):
    return pl.pallas_call(
        _router_body,
        out_shape=[jax.ShapeDtypeStruct((S, K), jnp.float32),
                   jax.ShapeDtypeStruct((S, K), jnp.int32),
                   jax.ShapeDtypeStruct((64, 1), jnp.int32)],
    )(x, wr)


_SP = P // 32


def _dispatch_body(dest_ref, wf_ref, x_ref, xr_ref, gate_ref,
                   d_v, w_v, t_v, rows_v, sem, sem2):
    cid = lax.axis_index("c")
    sid = lax.axis_index("s")
    w = sid * 2 + cid
    base = w * _SP
    pltpu.sync_copy(dest_ref.at[pl.ds(base, _SP)], d_v)
    pltpu.sync_copy(wf_ref.at[pl.ds(base, _SP)], w_v)
    lanes = lax.iota(jnp.int32, L)
    for j in range(_SP // L):
        tok = lax.shift_right_logical(base + j * L + lanes, 1)
        t_v[pl.ds(j * L, L)] = tok
    gate_cp = pltpu.async_copy(w_v, gate_ref.at[d_v], sem2)
    pltpu.async_copy(x_ref.at[t_v], rows_v, sem).wait()
    pltpu.async_copy(rows_v, xr_ref.at[d_v], sem).wait()
    gate_cp.wait()


_dispatch = functools.partial(
    pl.kernel,
    out_type=[jax.ShapeDtypeStruct((NSLOT, D), jnp.float32),
              jax.ShapeDtypeStruct((NSLOT,), jnp.float32)],
    mesh=_mesh,
    scratch_types=[pltpu.VMEM((_SP,), jnp.int32),
                   pltpu.VMEM((_SP,), jnp.float32),
                   pltpu.VMEM((_SP,), jnp.int32),
                   pltpu.VMEM((_SP, D), jnp.float32),
                   pltpu.SemaphoreType.DMA,
                   pltpu.SemaphoreType.DMA],
)(_dispatch_body)


def _ffn_body(bm_ref, xr_ref, g_ref, w1_ref, b1_ref, w2_ref, b2_ref, y_ref):
    b = pl.program_id(0)

    @pl.when(b < bm_ref[32])
    def _go():
        h = jnp.dot(xr_ref[...], w1_ref[0],
                    preferred_element_type=jnp.float32) + b1_ref[0]
        h = 0.5 * h * (1.0 + lax.erf(h / _SQRT2))
        y = jnp.dot(h, w2_ref[0], preferred_element_type=jnp.float32)
        y_ref[...] = g_ref[...] * (y + b2_ref[0])


def _ffn(bm, xr, gates, w1, b1, w2, b2):
    grid_spec = pltpu.PrefetchScalarGridSpec(
        num_scalar_prefetch=1,
        grid=(NB,),
        in_specs=[
            pl.BlockSpec((CBLK, D), lambda b, bm: (b, 0)),
            pl.BlockSpec((CBLK, 1), lambda b, bm: (b, 0)),
            pl.BlockSpec((1, D, FF), lambda b, bm: (bm[b], 0, 0)),
            pl.BlockSpec((1, 1, FF), lambda b, bm: (bm[b], 0, 0)),
            pl.BlockSpec((1, FF, D), lambda b, bm: (bm[b], 0, 0)),
            pl.BlockSpec((1, 1, D), lambda b, bm: (bm[b], 0, 0)),
        ],
        out_specs=pl.BlockSpec((CBLK, D), lambda b, bm: (b, 0)),
    )
    return pl.pallas_call(
        _ffn_body,
        grid_spec=grid_spec,
        out_shape=jax.ShapeDtypeStruct((NSLOT, D), jnp.float32),
    )(bm, xr, gates, w1, b1.reshape(E, 1, FF), w2, b2.reshape(E, 1, D))


_CT = S // 32
_CC = 32


def _combine_body(pos_ref, y_ref, out_ref, idx_v, rows_v, o_v, sem):
    cid = lax.axis_index("c")
    sid = lax.axis_index("s")
    w = sid * 2 + cid
    for c in range(_CT // _CC):
        t0 = w * _CT + c * _CC
        pltpu.sync_copy(pos_ref.at[pl.ds(t0 * K, _CC * K)], idx_v)
        pltpu.async_copy(y_ref.at[idx_v], rows_v, sem).wait()

        def _row(j, _):
            for l in range(D // L):
                a = rows_v[2 * j, pl.ds(l * L, L)]
                b = rows_v[2 * j + 1, pl.ds(l * L, L)]
                o_v[j, pl.ds(l * L, L)] = a + b
            return 0

        lax.fori_loop(0, _CC, _row, 0)
        pltpu.sync_copy(o_v, out_ref.at[pl.ds(t0, _CC)])


_combine = functools.partial(
    pl.kernel,
    out_type=jax.ShapeDtypeStruct((S, D), jnp.float32),
    mesh=_mesh,
    scratch_types=[pltpu.VMEM((_CC * K,), jnp.int32),
                   pltpu.VMEM((_CC * K, D), jnp.float32),
                   pltpu.VMEM((_CC, D), jnp.float32),
                   pltpu.SemaphoreType.DMA],
)(_combine_body)


def kernel(hidden_states, W_router, W1, b1, W2, b2):
    x = hidden_states.reshape(S, D)
    wts, dest, bm = _router(x, W_router)
    destf = dest.reshape(P)
    xr, gate = _dispatch(destf, wts.reshape(P), x)
    y = _ffn(bm.reshape(64), xr, gate.reshape(NSLOT, 1), W1, b1, W2, b2)
    out = _combine(destf, y)
    return out.reshape(1, S, D)
